# Initial kernel scaffold; baseline (speedup 1.0000x reference)
import jax
import jax.numpy as jnp
from jax.experimental import pallas as pl


def kernel(x, tables):
    raise NotImplementedError("write your pallas kernel here")


# speedup vs baseline: 1.0080x; 1.0080x over previous
import jax
import jax.numpy as jnp
import numpy as np
from jax import lax
from jax.experimental import pallas as pl
from jax.experimental.pallas import tpu as pltpu
from jax.experimental.pallas import tpu_sc as plsc

_LEVELS = 16
_FEATURES = 2
_TABLE = 524288
_MASK = _TABLE - 1
_BASE_RES = 16
_SCALE = 1.5
_N = 262144
_P1 = np.int32(73856093)
_P2 = np.int32(19349663)
_P3 = np.int32(83492791)
_RES = [int(_BASE_RES * _SCALE**l) for l in range(_LEVELS)]

_NC, _NS = 2, 16
_NW = _NC * _NS
_C = 128
_PPW = _N // _NW
_NCHUNK = _PPW // _C


def _splat(v):
    return jnp.full((16,), v, dtype=jnp.int32)


def _loop32(n):
    def deco(body):
        def wrapped(i, carry):
            body(i)
            return carry
        return lax.fori_loop(jnp.int32(0), jnp.int32(n), wrapped, None)
    return deco


def _hash_grid_body(xt_hbm, tab_hbm, out_hbm, xt_v, idx_v, gbuf, out_v, sems):
    wid = (
        lax.axis_index("s").astype(jnp.int32) * np.int32(_NC)
        + lax.axis_index("c").astype(jnp.int32)
    )

    @_loop32(_NCHUNK)
    def _chunk(g):
        g = g.astype(jnp.int32)
        base = (wid * np.int32(_NCHUNK) + g) * np.int32(_C)
        pltpu.sync_copy(xt_hbm.at[:, pl.ds(base, _C)], xt_v)

        def compute_idx(l):
            res = np.float32(_RES[l])

            @_loop32(_C // 16)
            def _indices(i):
                i = i.astype(jnp.int32)
                off = i * np.int32(16)
                xv = xt_v[np.int32(0), pl.ds(off, 16)]
                yv = xt_v[np.int32(1), pl.ds(off, 16)]
                zv = xt_v[np.int32(2), pl.ds(off, 16)]
                xi = (xv * res).astype(jnp.int32)
                yi = (yv * res).astype(jnp.int32)
                zi = (zv * res).astype(jnp.int32)
                one = np.int32(1)
                hx = (xi * _P1, (xi + one) * _P1)
                hy = (yi * _P2, (yi + one) * _P2)
                hz = (zi * _P3, (zi + one) * _P3)
                for c in range(8):
                    h = hx[(c >> 2) & 1] ^ hy[(c >> 1) & 1] ^ hz[c & 1]
                    flat = ((h & np.int32(_MASK)) + np.int32(l * _TABLE)) * np.int32(2)
                    idx_v[np.int32(l), np.int32(c), np.int32(0), pl.ds(off, 16)] = flat
                    idx_v[np.int32(l), np.int32(c), np.int32(1), pl.ds(off, 16)] = (
                        flat + one
                    )

        def fire(l):
            return [
                pltpu.async_copy(
                    tab_hbm.at[idx_v.at[np.int32(l), np.int32(c), np.int32(f)]],
                    gbuf.at[np.int32(l), np.int32(c), np.int32(f)],
                    sems.at[np.int32(l)],
                )
                for c in range(8)
                for f in range(_FEATURES)
            ]

        def interp(l):
            res = np.float32(_RES[l])

            @_loop32(_C // 16)
            def _interp(i):
                i = i.astype(jnp.int32)
                off = i * np.int32(16)
                pidx = off + lax.iota(jnp.int32, 16)
                xv = xt_v[np.int32(0), pl.ds(off, 16)]
                yv = xt_v[np.int32(1), pl.ds(off, 16)]
                zv = xt_v[np.int32(2), pl.ds(off, 16)]
                px = xv * res
                py = yv * res
                pz = zv * res
                wx = px - px.astype(jnp.int32).astype(jnp.float32)
                wy = py - py.astype(jnp.int32).astype(jnp.float32)
                wz = pz - pz.astype(jnp.int32).astype(jnp.float32)
                onef = np.float32(1.0)
                for f in range(_FEATURES):
                    v = [
                        gbuf[np.int32(l), np.int32(c), np.int32(f), pl.ds(off, 16)]
                        for c in range(8)
                    ]
                    v00 = v[0] * (onef - wz) + v[1] * wz
                    v01 = v[2] * (onef - wz) + v[3] * wz
                    v10 = v[4] * (onef - wz) + v[5] * wz
                    v11 = v[6] * (onef - wz) + v[7] * wz
                    v0 = v00 * (onef - wy) + v01 * wy
                    v1 = v10 * (onef - wy) + v11 * wy
                    emb = v0 * (onef - wx) + v1 * wx
                    plsc.store_scatter(out_v, [pidx, _splat(2 * l + f)], emb)

        handles = {}
        for l in range(_LEVELS):
            compute_idx(l)
            handles[l] = fire(l)
        for l in range(_LEVELS):
            for h in handles.pop(l):
                h.wait()
            interp(l)

        pltpu.sync_copy(out_v, out_hbm.at[pl.ds(base, _C)])


def kernel(x, tables):
    xt = x.T
    tab = tables.reshape(_LEVELS * _TABLE * _FEATURES)
    run = pl.kernel(
        _hash_grid_body,
        out_type=jax.ShapeDtypeStruct((_N, _LEVELS * _FEATURES), jnp.float32),
        mesh=plsc.VectorSubcoreMesh(
            core_axis_name="c", subcore_axis_name="s",
            num_cores=_NC, num_subcores=_NS,
        ),
        scratch_types=[
            pltpu.VMEM((3, _C), jnp.float32),
            pltpu.VMEM((_LEVELS, 8, _FEATURES, _C), jnp.int32),
            pltpu.VMEM((_LEVELS, 8, _FEATURES, _C), jnp.float32),
            pltpu.VMEM((_C, _LEVELS * _FEATURES), jnp.float32),
            pltpu.SemaphoreType.DMA((_LEVELS,)),
        ],
        compiler_params=pltpu.CompilerParams(
            needs_layout_passes=False, use_tc_tiling_on_sc=False,
        ),
    )
    return run(xt, tab)
